# baseline (speedup 1.0000x reference)
import functools

import numpy as np
import jax
import jax.numpy as jnp
from jax import lax
from jax.experimental import pallas as pl
from jax.experimental.pallas import tpu as pltpu
from jax.experimental.pallas import tpu_sc as plsc

_VOCAB = 100000
_D = 768
_S = 4096
_B = 4

_NC = 2
_NS = 16
_NW = _NC * _NS
_SPW = _S // _NW
_CS = 32
_NCH = _SPW // _CS
_DL = _D // 16


def _pos_encoding() -> np.ndarray:
    pos = np.arange(_S, dtype=np.float32)[:, None]
    i = np.arange(0, _D, 2, dtype=np.float32)
    div = np.exp(i * np.float32(-np.log(10000.0) / _D))
    ang = pos * div[None, :]
    pe = np.zeros((_S, _D), dtype=np.float32)
    pe[:, 0::2] = np.sin(ang)
    pe[:, 1::2] = np.cos(ang)
    return pe


_POS_NP = _pos_encoding()

_mesh = plsc.VectorSubcoreMesh(core_axis_name="c", subcore_axis_name="s")


@functools.partial(
    pl.kernel,
    mesh=_mesh,
    out_type=jax.ShapeDtypeStruct((_B, _S, _D), jnp.float32),
    scratch_types=[
        pltpu.VMEM((_B, _SPW), jnp.int32),
        pltpu.VMEM((_CS, _D), jnp.float32),
        pltpu.VMEM((_CS, _D), jnp.float32),
        pltpu.VMEM((_CS, _D), jnp.float32),
        pltpu.VMEM((_CS, _D), jnp.float32),
        pltpu.VMEM((_CS, _D), jnp.float32),
        pltpu.SemaphoreType.DMA,
        pltpu.SemaphoreType.DMA,
        pltpu.SemaphoreType.DMA,
        pltpu.SemaphoreType.DMA,
        pltpu.SemaphoreType.DMA,
        pltpu.SemaphoreType.DMA,
        pltpu.SemaphoreType.DMA,
        pltpu.SemaphoreType.DMA,
        pltpu.SemaphoreType.DMA,
        pltpu.SemaphoreType.DMA,
        pltpu.SemaphoreType.DMA,
        pltpu.SemaphoreType.DMA,
        pltpu.SemaphoreType.DMA,
        pltpu.SemaphoreType.DMA,
        pltpu.SemaphoreType.DMA,
        pltpu.SemaphoreType.DMA,
    ],
)
def _emb_kernel(x_hbm, table_hbm, pos_hbm, out_hbm,
                idx_all, pos_v, rows0, rows1, rows2, rows3,
                ga0, ga1, ga2, ga3, gb0, gb1, gb2, gb3,
                sa0, sa1, sa2, sa3, sb0, sb1, sb2, sb3):
    wid = lax.axis_index("s") * _NC + lax.axis_index("c")
    base = wid * _SPW
    pltpu.sync_copy(x_hbm.at[:, pl.ds(base, _SPW)], idx_all)
    rows = (rows0, rows1, rows2, rows3)
    gsem = ((ga0, ga1, ga2, ga3), (gb0, gb1, gb2, gb3))
    ssem = ((sa0, sa1, sa2, sa3), (sb0, sb1, sb2, sb3))
    NB = 4
    _H = _CS // 2
    NT = _NCH * _B

    def gather_start(t, h):
        ch, b = divmod(t, _B)
        return pltpu.async_copy(
            table_hbm.at[idx_all.at[b, pl.ds(ch * _CS + h * _H, _H)]],
            rows[t % NB].at[pl.ds(h * _H, _H)], gsem[h][t % NB])

    def store_start(t, h):
        ch, b = divmod(t, _B)
        return pltpu.async_copy(
            rows[t % NB].at[pl.ds(h * _H, _H)],
            out_hbm.at[b, pl.ds(base + ch * _CS + h * _H, _H)],
            ssem[h][t % NB])

    g_desc = [[None, None] for _ in range(NT + 2)]
    s_desc = [[None, None] for _ in range(NT)]
    for tt in (0, 1):
        g_desc[tt][0] = gather_start(tt, 0)
        g_desc[tt][1] = gather_start(tt, 1)
    for t in range(NT):
        ch, b = divmod(t, _B)
        if b == 0:
            pltpu.sync_copy(pos_hbm.at[pl.ds(base + ch * _CS, _CS)], pos_v)
        if t >= 2:
            s_desc[t - 2][0].wait()
            s_desc[t - 2][1].wait()
        if t + 2 < NT:
            g_desc[t + 2][0] = gather_start(t + 2, 0)
            g_desc[t + 2][1] = gather_start(t + 2, 1)
        buf = rows[t % NB]
        for h in (0, 1):
            g_desc[t][h].wait()
            hbase = h * _H

            @plsc.parallel_loop(0, _H, 1)
            def _add(r):
                @plsc.parallel_loop(0, _DL * 16, 16, unroll=8)
                def _addk(k):
                    sl = pl.ds(k, 16)
                    buf[hbase + r, sl] = buf[hbase + r, sl] + pos_v[hbase + r, sl]

            s_desc[t][h] = store_start(t, h)
    for t in (NT - 2, NT - 1):
        s_desc[t][0].wait()
        s_desc[t][1].wait()


def kernel(x, table):
    return _emb_kernel(x.astype(jnp.int32), table, jnp.asarray(_POS_NP))
